# Initial kernel scaffold; baseline (speedup 1.0000x reference)
import functools

import jax
import jax.numpy as jnp
from jax import lax
from jax.experimental import pallas as pl
from jax.experimental.pallas import tpu as pltpu
from jax.experimental.pallas import tpu_sc as plsc

VOCAB = 100000
HIDDEN = 128
B, L = 1024, 200
N_TOK = B * L
NC, NS, LANES = 2, 16, 16
NW = NC * NS
TOK_PER_W = N_TOK // NW
CHUNK = 128
NCHUNK = TOK_PER_W // CHUNK
NVREG = HIDDEN // LANES


def _rsqrt(v):
    i = lax.bitcast_convert_type(v, jnp.int32)
    y = lax.bitcast_convert_type(jnp.int32(0x5F3759DF) - (i >> 1), jnp.float32)
    for _ in range(3):
        y = y * (1.5 - 0.5 * v * y * y)
    return y


def _body(ids_hbm, tids_hbm, table_hbm, seg_hbm, gam_hbm, bet_hbm, out_hbm,
          idx_v, tid_v, rows_v, seg_v, gam_v, bet_v, gsem):
    wid = lax.axis_index("s") * NC + lax.axis_index("c")
    w_base = wid * TOK_PER_W

    pltpu.sync_copy(seg_hbm, seg_v)
    pltpu.sync_copy(gam_hbm, gam_v)
    pltpu.sync_copy(bet_hbm, bet_v)

    def chunk_body(c, carry):
        base = w_base + c * CHUNK
        pltpu.sync_copy(ids_hbm.at[pl.ds(base, CHUNK)], idx_v)
        pltpu.sync_copy(tids_hbm.at[pl.ds(base, CHUNK)], tid_v)
        pltpu.async_copy(table_hbm.at[idx_v], rows_v, gsem).wait()

        def tok_body(t, carry2):
            tb = plsc.load_gather(tid_v, [jnp.full((LANES,), t, jnp.int32)])
            pb = tb > 0
            x = [rows_v[t, pl.ds(j * LANES, LANES)] for j in range(NVREG)]
            for j in range(NVREG):
                s0 = seg_v[0, pl.ds(j * LANES, LANES)]
                s1 = seg_v[1, pl.ds(j * LANES, LANES)]
                x[j] = x[j] + jnp.where(pb, s1, s0)
            tot = x[0]
            for j in range(1, NVREG):
                tot = tot + x[j]
            mu = jnp.sum(tot) * (1.0 / HIDDEN)
            mu_b = jnp.full((LANES,), mu)
            sq = (x[0] - mu_b) * (x[0] - mu_b)
            for j in range(1, NVREG):
                d = x[j] - mu_b
                sq = sq + d * d
            var = jnp.sum(sq) * (1.0 / HIDDEN)
            r_b = _rsqrt(jnp.full((LANES,), var + 1e-12))
            for j in range(NVREG):
                g = gam_v[pl.ds(j * LANES, LANES)]
                bt = bet_v[pl.ds(j * LANES, LANES)]
                rows_v[t, pl.ds(j * LANES, LANES)] = (x[j] - mu_b) * r_b * g + bt
            return carry2

        lax.fori_loop(0, CHUNK, tok_body, 0)
        pltpu.sync_copy(rows_v, out_hbm.at[pl.ds(base, CHUNK)])
        return carry

    lax.fori_loop(0, NCHUNK, chunk_body, 0)


_mesh = plsc.VectorSubcoreMesh(core_axis_name="c", subcore_axis_name="s")

_sc_call = functools.partial(
    pl.kernel,
    mesh=_mesh,
    out_type=jax.ShapeDtypeStruct((N_TOK, HIDDEN), jnp.float32),
    scratch_types=[
        pltpu.VMEM((CHUNK,), jnp.int32),
        pltpu.VMEM((CHUNK,), jnp.int32),
        pltpu.VMEM((CHUNK, HIDDEN), jnp.float32),
        pltpu.VMEM((2, HIDDEN), jnp.float32),
        pltpu.VMEM((HIDDEN,), jnp.float32),
        pltpu.VMEM((HIDDEN,), jnp.float32),
        pltpu.SemaphoreType.DMA,
    ],
)(_body)


def kernel(input_ids, token_type_ids, word_embeddings, segment_embeddings, ln_gamma, ln_beta):
    ids = input_ids.reshape(-1).astype(jnp.int32)
    tids = token_type_ids.reshape(-1).astype(jnp.int32)
    out = _sc_call(ids, tids, word_embeddings, segment_embeddings, ln_gamma, ln_beta)
    return out.reshape(B, L, HIDDEN)


# speedup vs baseline: 1.2274x; 1.2274x over previous
import functools

import jax
import jax.numpy as jnp
from jax import lax
from jax.experimental import pallas as pl
from jax.experimental.pallas import tpu as pltpu
from jax.experimental.pallas import tpu_sc as plsc

VOCAB = 100000
HIDDEN = 128
B, L = 1024, 200
N_TOK = B * L
NC, NS, LANES = 2, 16, 16
NW = NC * NS
TOK_PER_W = N_TOK // NW
CHUNK = 128
NCHUNK = TOK_PER_W // CHUNK
NVREG = HIDDEN // LANES


def _rsqrt(v):
    i = lax.bitcast_convert_type(v, jnp.int32)
    y = lax.bitcast_convert_type(jnp.int32(0x5F3759DF) - (i >> 1), jnp.float32)
    for _ in range(3):
        y = y * (1.5 - 0.5 * v * y * y)
    return y


def _body(ids_hbm, tids_hbm, table_hbm, seg_hbm, gam_hbm, bet_hbm, out_hbm,
          idx_v, tid_v, rows_v, seg_v, gam_v, bet_v, gsem):
    wid = lax.axis_index("s") * NC + lax.axis_index("c")
    w_base = wid * TOK_PER_W

    pltpu.sync_copy(seg_hbm, seg_v)
    pltpu.sync_copy(gam_hbm, gam_v)
    pltpu.sync_copy(bet_hbm, bet_v)

    def chunk_body(c, carry):
        base = w_base + c * CHUNK
        pltpu.sync_copy(ids_hbm.at[pl.ds(base, CHUNK)], idx_v)
        pltpu.sync_copy(tids_hbm.at[pl.ds(base, CHUNK)], tid_v)
        pltpu.async_copy(table_hbm.at[idx_v], rows_v, gsem).wait()

        def grp_body(g, carry2):
            tg = tid_v[pl.ds(g * LANES, LANES)]
            for k in range(LANES):
                t = g * LANES + k
                pb = jnp.full((LANES,), tg[k] > 0)
                x = [rows_v[t, pl.ds(j * LANES, LANES)] for j in range(NVREG)]
                for j in range(NVREG):
                    s0 = seg_v[0, pl.ds(j * LANES, LANES)]
                    s1 = seg_v[1, pl.ds(j * LANES, LANES)]
                    x[j] = x[j] + jnp.where(pb, s1, s0)
                tot = x[0]
                for j in range(1, NVREG):
                    tot = tot + x[j]
                mu = jnp.sum(tot) * (1.0 / HIDDEN)
                mu_b = jnp.full((LANES,), mu)
                sq = (x[0] - mu_b) * (x[0] - mu_b)
                for j in range(1, NVREG):
                    d = x[j] - mu_b
                    sq = sq + d * d
                var = jnp.sum(sq) * (1.0 / HIDDEN)
                r_b = _rsqrt(jnp.full((LANES,), var + 1e-12))
                for j in range(NVREG):
                    gm = gam_v[pl.ds(j * LANES, LANES)]
                    bt = bet_v[pl.ds(j * LANES, LANES)]
                    rows_v[t, pl.ds(j * LANES, LANES)] = (x[j] - mu_b) * r_b * gm + bt
            return carry2

        lax.fori_loop(0, CHUNK // LANES, grp_body, 0)
        pltpu.sync_copy(rows_v, out_hbm.at[pl.ds(base, CHUNK)])
        return carry

    lax.fori_loop(0, NCHUNK, chunk_body, 0)


_mesh = plsc.VectorSubcoreMesh(core_axis_name="c", subcore_axis_name="s")

_sc_call = functools.partial(
    pl.kernel,
    mesh=_mesh,
    out_type=jax.ShapeDtypeStruct((N_TOK, HIDDEN), jnp.float32),
    scratch_types=[
        pltpu.VMEM((CHUNK,), jnp.int32),
        pltpu.VMEM((CHUNK,), jnp.int32),
        pltpu.VMEM((CHUNK, HIDDEN), jnp.float32),
        pltpu.VMEM((2, HIDDEN), jnp.float32),
        pltpu.VMEM((HIDDEN,), jnp.float32),
        pltpu.VMEM((HIDDEN,), jnp.float32),
        pltpu.SemaphoreType.DMA,
    ],
    compiler_params=pltpu.CompilerParams(needs_layout_passes=False),
)(_body)


def kernel(input_ids, token_type_ids, word_embeddings, segment_embeddings, ln_gamma, ln_beta):
    ids = input_ids.reshape(-1).astype(jnp.int32)
    tids = token_type_ids.reshape(-1).astype(jnp.int32)
    out = _sc_call(ids, tids, word_embeddings, segment_embeddings, ln_gamma, ln_beta)
    return out.reshape(B, L, HIDDEN)
